# baseline (speedup 1.0000x reference)
import jax
import jax.numpy as jnp
from jax.experimental import pallas as pl
from jax.experimental.pallas import tpu as pltpu

B = 4
S = 4096
D = 1024
CH = 1024
N_C = S // CH
T = N_C * B
NBUF = 8


def _body(x_hbm, emb_hbm, out_hbm, x_bufs, emb_bufs, in_sem, out_sem, emb_sem):
    def start_in(t):
        c, b, buf = t // B, t % B, t % NBUF
        pltpu.make_async_copy(x_hbm.at[b, pl.ds(c * CH, CH)], x_bufs.at[buf],
                              in_sem.at[buf]).start()

    def wait_in(t):
        c, b, buf = t // B, t % B, t % NBUF
        pltpu.make_async_copy(x_hbm.at[b, pl.ds(c * CH, CH)], x_bufs.at[buf],
                              in_sem.at[buf]).wait()

    def start_out(t):
        c, b, buf = t // B, t % B, t % NBUF
        pltpu.make_async_copy(x_bufs.at[buf], out_hbm.at[b, pl.ds(c * CH, CH)],
                              out_sem.at[buf]).start()

    def wait_out(t):
        c, b, buf = t // B, t % B, t % NBUF
        pltpu.make_async_copy(x_bufs.at[buf], out_hbm.at[b, pl.ds(c * CH, CH)],
                              out_sem.at[buf]).wait()

    def start_emb(c):
        pltpu.make_async_copy(emb_hbm.at[pl.ds(c * CH, CH)], emb_bufs.at[c % 2],
                              emb_sem.at[c % 2]).start()

    def wait_emb(c):
        pltpu.make_async_copy(emb_hbm.at[pl.ds(c * CH, CH)], emb_bufs.at[c % 2],
                              emb_sem.at[c % 2]).wait()

    start_emb(0)
    start_in(0)
    start_in(1)
    start_in(2)
    start_in(3)

    for t in range(T):
        c, b, buf = t // B, t % B, t % NBUF
        if t >= 4:
            wait_out(t - 4)
        if t + 4 < T:
            start_in(t + 4)
        if b == 0:
            if c + 1 < N_C:
                start_emb(c + 1)
            wait_emb(c)
        wait_in(t)
        x_bufs[buf] = x_bufs[buf] + emb_bufs[c % 2]
        start_out(t)

    wait_out(T - 4)
    wait_out(T - 3)
    wait_out(T - 2)
    wait_out(T - 1)


@jax.jit
def kernel(x, emb):
    out = pl.pallas_call(
        _body,
        in_specs=[
            pl.BlockSpec(memory_space=pl.ANY),
            pl.BlockSpec(memory_space=pl.ANY),
        ],
        out_specs=pl.BlockSpec(memory_space=pl.ANY),
        out_shape=jax.ShapeDtypeStruct((B, S, D), jnp.float32),
        scratch_shapes=[
            pltpu.VMEM((NBUF, CH, D), jnp.float32),
            pltpu.VMEM((2, CH, D), jnp.float32),
            pltpu.SemaphoreType.DMA((NBUF,)),
            pltpu.SemaphoreType.DMA((NBUF,)),
            pltpu.SemaphoreType.DMA((2,)),
        ],
    )(x, emb)
    return out


# speedup vs baseline: 1.0128x; 1.0128x over previous
import jax
import jax.numpy as jnp
from jax.experimental import pallas as pl
from jax.experimental.pallas import tpu as pltpu

B = 4
S = 4096
D = 1024
CH = 1024
N_C = S // CH
T = N_C * B
NBUF = 6


def _body(x_hbm, emb_hbm, out_hbm, x_bufs, emb_bufs, in_sem, out_sem, emb_sem):
    def start_in(t):
        c, b, buf = t // B, t % B, t % NBUF
        pltpu.make_async_copy(x_hbm.at[b, pl.ds(c * CH, CH)], x_bufs.at[buf],
                              in_sem.at[buf]).start()

    def wait_in(t):
        c, b, buf = t // B, t % B, t % NBUF
        pltpu.make_async_copy(x_hbm.at[b, pl.ds(c * CH, CH)], x_bufs.at[buf],
                              in_sem.at[buf]).wait()

    def start_out(t):
        c, b, buf = t // B, t % B, t % NBUF
        pltpu.make_async_copy(x_bufs.at[buf], out_hbm.at[b, pl.ds(c * CH, CH)],
                              out_sem.at[buf]).start()

    def wait_out(t):
        c, b, buf = t // B, t % B, t % NBUF
        pltpu.make_async_copy(x_bufs.at[buf], out_hbm.at[b, pl.ds(c * CH, CH)],
                              out_sem.at[buf]).wait()

    def start_emb(c):
        pltpu.make_async_copy(emb_hbm.at[pl.ds(c * CH, CH)], emb_bufs.at[c % 2],
                              emb_sem.at[c % 2]).start()

    def wait_emb(c):
        pltpu.make_async_copy(emb_hbm.at[pl.ds(c * CH, CH)], emb_bufs.at[c % 2],
                              emb_sem.at[c % 2]).wait()

    start_emb(0)
    start_in(0)
    start_in(1)
    start_in(2)

    for t in range(T):
        c, b, buf = t // B, t % B, t % NBUF
        if t >= 3:
            wait_out(t - 3)
        if t + 3 < T:
            start_in(t + 3)
        if b == 0:
            if c + 1 < N_C:
                start_emb(c + 1)
            wait_emb(c)
        wait_in(t)
        x_bufs[buf] = x_bufs[buf] + emb_bufs[c % 2]
        start_out(t)

    wait_out(T - 3)
    wait_out(T - 2)
    wait_out(T - 1)


@jax.jit
def kernel(x, emb):
    out = pl.pallas_call(
        _body,
        in_specs=[
            pl.BlockSpec(memory_space=pl.ANY),
            pl.BlockSpec(memory_space=pl.ANY),
        ],
        out_specs=pl.BlockSpec(memory_space=pl.ANY),
        out_shape=jax.ShapeDtypeStruct((B, S, D), jnp.float32),
        scratch_shapes=[
            pltpu.VMEM((NBUF, CH, D), jnp.float32),
            pltpu.VMEM((2, CH, D), jnp.float32),
            pltpu.SemaphoreType.DMA((NBUF,)),
            pltpu.SemaphoreType.DMA((NBUF,)),
            pltpu.SemaphoreType.DMA((2,)),
        ],
    )(x, emb)
    return out
